# baseline (speedup 1.0000x reference)
import jax
import jax.numpy as jnp
from jax.experimental import pallas as pl
from jax.experimental.pallas import tpu as pltpu
from jax.experimental.pallas import tpu_sc as plsc

_ROWS = 256
_GATHER_WINDOW = 128


def _layer_body(x_ref, adj_ref, wt_ref, wb_ref, out_ref):
    r = pl.program_id(1)
    adj_s = adj_ref[0]
    x = x_ref[0]
    deg = jnp.sum(adj_s, axis=1, keepdims=True) + 1e-6
    xb = x.astype(jnp.bfloat16)
    neigh = jnp.dot(adj_s.astype(jnp.bfloat16), xb,
                    preferred_element_type=jnp.float32) / deg
    xr = x_ref[0, pl.ds(r * _ROWS, _ROWS), :].astype(jnp.bfloat16)
    h = (jnp.dot(xr, wt_ref[...].astype(jnp.bfloat16),
                 preferred_element_type=jnp.float32)
         + jnp.dot(neigh.astype(jnp.bfloat16), wb_ref[...].astype(jnp.bfloat16),
                   preferred_element_type=jnp.float32))
    out_ref[0] = jnp.maximum(h, 0.0)


def _sage_layer(x, adj, wt, wb):
    b, n, d = x.shape
    h = wt.shape[1]
    return pl.pallas_call(
        _layer_body,
        grid=(b, n // _ROWS),
        in_specs=[
            pl.BlockSpec((1, n, d), lambda i, r: (i, 0, 0)),
            pl.BlockSpec((1, _ROWS, n), lambda i, r: (i, r, 0)),
            pl.BlockSpec((d, h), lambda i, r: (0, 0)),
            pl.BlockSpec((d, h), lambda i, r: (0, 0)),
        ],
        out_specs=pl.BlockSpec((1, _ROWS, h), lambda i, r: (i, r, 0)),
        out_shape=jax.ShapeDtypeStruct((b, n, h), jnp.float32),
    )(x, adj, wt, wb)


def _gather(h2_flat, idx):
    bn, d = h2_flat.shape
    mesh = plsc.VectorSubcoreMesh(core_axis_name="c", subcore_axis_name="s")

    @pl.kernel(out_type=jax.ShapeDtypeStruct((bn, d), h2_flat.dtype),
               mesh=mesh)
    def k(x_hbm, i_hbm, o_hbm):
        def body(i_vmem, o_vmem):
            pltpu.sync_copy(x_hbm.at[i_vmem.at[0]], o_vmem)

        pltpu.emit_pipeline(
            body,
            grid=(bn // _GATHER_WINDOW,),
            in_specs=[pl.BlockSpec((1, _GATHER_WINDOW), lambda i: (0, i))],
            out_specs=[pl.BlockSpec((_GATHER_WINDOW, d), lambda i: (i, 0))],
            core_axis_name=("c", "s"),
            dimension_semantics=(pltpu.PARALLEL,),
        )(i_hbm, o_hbm)

    return k(h2_flat, idx)


def _loss_body(emb_ref, adjm_ref, acc_ref):
    b = pl.program_id(0)
    r = pl.program_id(1)
    ef = emb_ref[0]
    er = emb_ref[0, pl.ds(r * _ROWS, _ROWS), :]
    z = jax.lax.dot_general(er, ef, (((1,), (1,)), ((), ())),
                            preferred_element_type=jnp.float32)
    a = adjm_ref[0]
    s = 0.5 * (jnp.tanh(0.5 * z) + 1.0)
    t = a * jnp.log(s + 1e-7) + (1.0 - a) * jnp.log(1.0 - s + 1e-7)
    part = jnp.sum(t)

    @pl.when((b == 0) & (r == 0))
    def _():
        acc_ref[...] = jnp.zeros_like(acc_ref)

    acc_ref[...] += part


def _loss_call(embed, adjm):
    b, n, d = embed.shape
    return pl.pallas_call(
        _loss_body,
        grid=(b, n // _ROWS),
        in_specs=[
            pl.BlockSpec((1, n, d), lambda i, r: (i, 0, 0)),
            pl.BlockSpec((1, _ROWS, n), lambda i, r: (i, r, 0)),
        ],
        out_specs=pl.BlockSpec((8, 128), lambda i, r: (0, 0)),
        out_shape=jax.ShapeDtypeStruct((8, 128), jnp.float32),
    )(embed, adjm)


def _head_body(emb_ref, acc_ref, w1_ref, b1_ref, w2_ref, b2_ref, w3_ref,
               b3_ref, v_ref, loss_ref):
    bsz, n, _ = emb_ref.shape
    means = jnp.mean(emb_ref[...], axis=1)
    x1 = jnp.maximum(
        jnp.dot(means, w1_ref[...], preferred_element_type=jnp.float32)
        + b1_ref[...], 0.0)
    x2 = jnp.maximum(
        jnp.dot(x1, w2_ref[...], preferred_element_type=jnp.float32)
        + b2_ref[...], 0.0)
    v = jnp.sum(x2 * w3_ref[...], axis=1, keepdims=True) + b3_ref[...]
    v_ref[...] = jnp.broadcast_to(v, v_ref.shape)
    loss_ref[...] = jnp.full(loss_ref.shape,
                             -acc_ref[0, 0] / (bsz * n * n), jnp.float32)


def _head_call(embed, acc, w1, b1, w2, b2, w3, b3):
    b = embed.shape[0]
    return pl.pallas_call(
        _head_body,
        out_shape=(jax.ShapeDtypeStruct((b, 128), jnp.float32),
                   jax.ShapeDtypeStruct((8, 128), jnp.float32)),
    )(embed, acc, w1, b1, w2, b2, w3, b3)


def kernel(features, adj_lists, nodes, adj_matrixs, W1, W2,
           fc1_w, fc1_b, fc2_w, fc2_b, fc3_w, fc3_b):
    b, n, d = features.shape
    h = W1.shape[1]
    out_d = W2.shape[1]

    h1 = _sage_layer(features, adj_lists, W1[:d], W1[d:])
    h2 = _sage_layer(h1, adj_lists, W2[:h], W2[h:])

    idx = (nodes.astype(jnp.int32)
           + (jnp.arange(b, dtype=jnp.int32) * n)[:, None]).reshape(1, b * n)
    embed = _gather(h2.reshape(b * n, out_d), idx).reshape(b, n, out_d)

    acc = _loss_call(embed, adj_matrixs)
    v_pad, loss_pad = _head_call(
        embed, acc, fc1_w, fc1_b.reshape(1, -1), fc2_w, fc2_b.reshape(1, -1),
        fc3_w.reshape(1, -1), fc3_b.reshape(1, 1))
    return (v_pad[:, :1], loss_pad[0, 0])


# speedup vs baseline: 1.2513x; 1.2513x over previous
import jax
import jax.numpy as jnp
from jax.experimental import pallas as pl
from jax.experimental.pallas import tpu as pltpu
from jax.experimental.pallas import tpu_sc as plsc

_ROWS = 256
_GATHER_WINDOW = 128


def _sage2_body(x_ref, adj_ref, w1t_ref, w1b_ref, w2t_ref, w2b_ref, out_ref):
    adj = adj_ref[0]
    x = x_ref[0]
    deg = jnp.sum(adj, axis=1, keepdims=True) + 1e-6
    neigh1 = jnp.dot(adj, x, preferred_element_type=jnp.float32) / deg
    h1 = jnp.maximum(
        jnp.dot(x, w1t_ref[...], preferred_element_type=jnp.float32)
        + jnp.dot(neigh1, w1b_ref[...], preferred_element_type=jnp.float32),
        0.0)
    neigh2 = jnp.dot(adj, h1, preferred_element_type=jnp.float32) / deg
    h2 = jnp.maximum(
        jnp.dot(h1, w2t_ref[...], preferred_element_type=jnp.float32)
        + jnp.dot(neigh2, w2b_ref[...], preferred_element_type=jnp.float32),
        0.0)
    out_ref[0] = h2


def _sage2(x, adj, w1t, w1b, w2t, w2b):
    b, n, d = x.shape
    h = w2t.shape[1]
    return pl.pallas_call(
        _sage2_body,
        grid=(b,),
        in_specs=[
            pl.BlockSpec((1, n, d), lambda i: (i, 0, 0)),
            pl.BlockSpec((1, n, n), lambda i: (i, 0, 0)),
            pl.BlockSpec(w1t.shape, lambda i: (0, 0)),
            pl.BlockSpec(w1b.shape, lambda i: (0, 0)),
            pl.BlockSpec(w2t.shape, lambda i: (0, 0)),
            pl.BlockSpec(w2b.shape, lambda i: (0, 0)),
        ],
        out_specs=pl.BlockSpec((1, n, h), lambda i: (i, 0, 0)),
        out_shape=jax.ShapeDtypeStruct((b, n, h), jnp.float32),
    )(x, adj, w1t, w1b, w2t, w2b)


def _gather(h2_flat, idx):
    bn, d = h2_flat.shape
    mesh = plsc.VectorSubcoreMesh(core_axis_name="c", subcore_axis_name="s")

    @pl.kernel(out_type=jax.ShapeDtypeStruct((bn, d), h2_flat.dtype),
               mesh=mesh)
    def k(x_hbm, i_hbm, o_hbm):
        def body(i_vmem, o_vmem):
            pltpu.sync_copy(x_hbm.at[i_vmem.at[0]], o_vmem)

        pltpu.emit_pipeline(
            body,
            grid=(bn // _GATHER_WINDOW,),
            in_specs=[pl.BlockSpec((1, _GATHER_WINDOW), lambda i: (0, i))],
            out_specs=[pl.BlockSpec((_GATHER_WINDOW, d), lambda i: (i, 0))],
            core_axis_name=("c", "s"),
            dimension_semantics=(pltpu.PARALLEL,),
        )(i_hbm, o_hbm)

    return k(h2_flat, idx)


def _loss_body(emb_ref, adjm_ref, acc_ref):
    b = pl.program_id(0)
    r = pl.program_id(1)
    ef = emb_ref[0]
    er = emb_ref[0, pl.ds(r * _ROWS, _ROWS), :]
    z = jax.lax.dot_general(er, ef, (((1,), (1,)), ((), ())),
                            preferred_element_type=jnp.float32)
    a = adjm_ref[0]
    s = 1.0 / (1.0 + jnp.exp(-z))
    t = a * jnp.log(s + 1e-7) + (1.0 - a) * jnp.log(1.0000001192092896 - s)
    part = jnp.sum(t)

    @pl.when((b == 0) & (r == 0))
    def _():
        acc_ref[...] = jnp.zeros_like(acc_ref)

    acc_ref[...] += part


def _loss_call(embed, adjm):
    b, n, d = embed.shape
    return pl.pallas_call(
        _loss_body,
        grid=(b, n // _ROWS),
        in_specs=[
            pl.BlockSpec((1, n, d), lambda i, r: (i, 0, 0)),
            pl.BlockSpec((1, _ROWS, n), lambda i, r: (i, r, 0)),
        ],
        out_specs=pl.BlockSpec((8, 128), lambda i, r: (0, 0)),
        out_shape=jax.ShapeDtypeStruct((8, 128), jnp.float32),
    )(embed, adjm)


def _head_body(emb_ref, acc_ref, w1_ref, b1_ref, w2_ref, b2_ref, w3_ref,
               b3_ref, v_ref, loss_ref):
    bsz, n, _ = emb_ref.shape
    means = jnp.mean(emb_ref[...], axis=1)
    x1 = jnp.maximum(
        jnp.dot(means, w1_ref[...], preferred_element_type=jnp.float32)
        + b1_ref[...], 0.0)
    x2 = jnp.maximum(
        jnp.dot(x1, w2_ref[...], preferred_element_type=jnp.float32)
        + b2_ref[...], 0.0)
    v = jnp.sum(x2 * w3_ref[...], axis=1, keepdims=True) + b3_ref[...]
    v_ref[...] = jnp.broadcast_to(v, v_ref.shape)
    loss_ref[...] = jnp.full(loss_ref.shape,
                             -acc_ref[0, 0] / (bsz * n * n), jnp.float32)


def _head_call(embed, acc, w1, b1, w2, b2, w3, b3):
    b = embed.shape[0]
    return pl.pallas_call(
        _head_body,
        out_shape=(jax.ShapeDtypeStruct((b, 128), jnp.float32),
                   jax.ShapeDtypeStruct((8, 128), jnp.float32)),
    )(embed, acc, w1, b1, w2, b2, w3, b3)


def kernel(features, adj_lists, nodes, adj_matrixs, W1, W2,
           fc1_w, fc1_b, fc2_w, fc2_b, fc3_w, fc3_b):
    b, n, d = features.shape
    h = W1.shape[1]
    out_d = W2.shape[1]

    h2 = _sage2(features, adj_lists, W1[:d], W1[d:], W2[:h], W2[h:])

    idx = (nodes.astype(jnp.int32)
           + (jnp.arange(b, dtype=jnp.int32) * n)[:, None]).reshape(1, b * n)
    embed = _gather(h2.reshape(b * n, out_d), idx).reshape(b, n, out_d)

    acc = _loss_call(embed, adj_matrixs)
    v_pad, loss_pad = _head_call(
        embed, acc, fc1_w, fc1_b.reshape(1, -1), fc2_w, fc2_b.reshape(1, -1),
        fc3_w.reshape(1, -1), fc3_b.reshape(1, 1))
    return (v_pad[:, :1], loss_pad[0, 0])


# speedup vs baseline: 1.2632x; 1.0095x over previous
import jax
import jax.numpy as jnp
from jax.experimental import pallas as pl
from jax.experimental.pallas import tpu as pltpu
from jax.experimental.pallas import tpu_sc as plsc

_ROWS = 256
_GATHER_WINDOW = 128


def _sage2_body(x_ref, adj_ref, w1t_ref, w1b_ref, w2t_ref, w2b_ref, out_ref):
    adj = adj_ref[0]
    x = x_ref[0]
    deg = jnp.sum(adj, axis=1, keepdims=True) + 1e-6
    adjb = adj.astype(jnp.bfloat16)
    xb = x.astype(jnp.bfloat16)
    neigh1 = jnp.dot(adjb, xb, preferred_element_type=jnp.float32) / deg
    h1 = jnp.maximum(
        jnp.dot(xb, w1t_ref[...].astype(jnp.bfloat16),
                preferred_element_type=jnp.float32)
        + jnp.dot(neigh1.astype(jnp.bfloat16),
                  w1b_ref[...].astype(jnp.bfloat16),
                  preferred_element_type=jnp.float32), 0.0)
    h1b = h1.astype(jnp.bfloat16)
    neigh2 = jnp.dot(adjb, h1b, preferred_element_type=jnp.float32) / deg
    h2 = jnp.maximum(
        jnp.dot(h1b, w2t_ref[...].astype(jnp.bfloat16),
                preferred_element_type=jnp.float32)
        + jnp.dot(neigh2.astype(jnp.bfloat16),
                  w2b_ref[...].astype(jnp.bfloat16),
                  preferred_element_type=jnp.float32), 0.0)
    out_ref[0] = h2


def _sage2(x, adj, w1t, w1b, w2t, w2b):
    b, n, d = x.shape
    h = w2t.shape[1]
    return pl.pallas_call(
        _sage2_body,
        grid=(b,),
        in_specs=[
            pl.BlockSpec((1, n, d), lambda i: (i, 0, 0)),
            pl.BlockSpec((1, n, n), lambda i: (i, 0, 0)),
            pl.BlockSpec(w1t.shape, lambda i: (0, 0)),
            pl.BlockSpec(w1b.shape, lambda i: (0, 0)),
            pl.BlockSpec(w2t.shape, lambda i: (0, 0)),
            pl.BlockSpec(w2b.shape, lambda i: (0, 0)),
        ],
        out_specs=pl.BlockSpec((1, n, h), lambda i: (i, 0, 0)),
        out_shape=jax.ShapeDtypeStruct((b, n, h), jnp.float32),
    )(x, adj, w1t, w1b, w2t, w2b)


def _gather(h2_flat, idx):
    bn, d = h2_flat.shape
    mesh = plsc.VectorSubcoreMesh(core_axis_name="c", subcore_axis_name="s")

    @pl.kernel(out_type=jax.ShapeDtypeStruct((bn, d), h2_flat.dtype),
               mesh=mesh)
    def k(x_hbm, i_hbm, o_hbm):
        def body(i_vmem, o_vmem):
            pltpu.sync_copy(x_hbm.at[i_vmem.at[0]], o_vmem)

        pltpu.emit_pipeline(
            body,
            grid=(bn // _GATHER_WINDOW,),
            in_specs=[pl.BlockSpec((1, _GATHER_WINDOW), lambda i: (0, i))],
            out_specs=[pl.BlockSpec((_GATHER_WINDOW, d), lambda i: (i, 0))],
            core_axis_name=("c", "s"),
            dimension_semantics=(pltpu.PARALLEL,),
        )(i_hbm, o_hbm)

    return k(h2_flat, idx)


def _loss_body(emb_ref, adjm_ref, acc_ref):
    b = pl.program_id(0)
    r = pl.program_id(1)
    ef = emb_ref[0]
    er = emb_ref[0, pl.ds(r * _ROWS, _ROWS), :]
    z = jax.lax.dot_general(er, ef, (((1,), (1,)), ((), ())),
                            preferred_element_type=jnp.float32)
    a = adjm_ref[0]
    s = 1.0 / (1.0 + jnp.exp(-z))
    t = a * jnp.log(s + 1e-7) + (1.0 - a) * jnp.log(1.0000001192092896 - s)
    part = jnp.sum(t)

    @pl.when((b == 0) & (r == 0))
    def _():
        acc_ref[...] = jnp.zeros_like(acc_ref)

    acc_ref[...] += part


def _loss_call(embed, adjm):
    b, n, d = embed.shape
    return pl.pallas_call(
        _loss_body,
        grid=(b, n // _ROWS),
        in_specs=[
            pl.BlockSpec((1, n, d), lambda i, r: (i, 0, 0)),
            pl.BlockSpec((1, _ROWS, n), lambda i, r: (i, r, 0)),
        ],
        out_specs=pl.BlockSpec((8, 128), lambda i, r: (0, 0)),
        out_shape=jax.ShapeDtypeStruct((8, 128), jnp.float32),
    )(embed, adjm)


def _head_body(emb_ref, acc_ref, w1_ref, b1_ref, w2_ref, b2_ref, w3_ref,
               b3_ref, v_ref, loss_ref):
    bsz, n, _ = emb_ref.shape
    means = jnp.mean(emb_ref[...], axis=1)
    x1 = jnp.maximum(
        jnp.dot(means, w1_ref[...], preferred_element_type=jnp.float32)
        + b1_ref[...], 0.0)
    x2 = jnp.maximum(
        jnp.dot(x1, w2_ref[...], preferred_element_type=jnp.float32)
        + b2_ref[...], 0.0)
    v = jnp.sum(x2 * w3_ref[...], axis=1, keepdims=True) + b3_ref[...]
    v_ref[...] = jnp.broadcast_to(v, v_ref.shape)
    loss_ref[...] = jnp.full(loss_ref.shape,
                             -acc_ref[0, 0] / (bsz * n * n), jnp.float32)


def _head_call(embed, acc, w1, b1, w2, b2, w3, b3):
    b = embed.shape[0]
    return pl.pallas_call(
        _head_body,
        out_shape=(jax.ShapeDtypeStruct((b, 128), jnp.float32),
                   jax.ShapeDtypeStruct((8, 128), jnp.float32)),
    )(embed, acc, w1, b1, w2, b2, w3, b3)


def kernel(features, adj_lists, nodes, adj_matrixs, W1, W2,
           fc1_w, fc1_b, fc2_w, fc2_b, fc3_w, fc3_b):
    b, n, d = features.shape
    h = W1.shape[1]
    out_d = W2.shape[1]

    h2 = _sage2(features, adj_lists, W1[:d], W1[d:], W2[:h], W2[h:])

    idx = (nodes.astype(jnp.int32)
           + (jnp.arange(b, dtype=jnp.int32) * n)[:, None]).reshape(1, b * n)
    embed = _gather(h2.reshape(b * n, out_d), idx).reshape(b, n, out_d)

    acc = _loss_call(embed, adj_matrixs)
    v_pad, loss_pad = _head_call(
        embed, acc, fc1_w, fc1_b.reshape(1, -1), fc2_w, fc2_b.reshape(1, -1),
        fc3_w.reshape(1, -1), fc3_b.reshape(1, 1))
    return (v_pad[:, :1], loss_pad[0, 0])


# speedup vs baseline: 1.3200x; 1.0450x over previous
import jax
import jax.numpy as jnp
from jax.experimental import pallas as pl
from jax.experimental.pallas import tpu as pltpu
from jax.experimental.pallas import tpu_sc as plsc

_ROWS = 256
_GATHER_WINDOW = 128


def _sage2_body(x_ref, adj_ref, w1_ref, w2_ref, out_ref):
    adj = adj_ref[0]
    x = x_ref[0]
    deg = jnp.sum(adj, axis=1, keepdims=True) + 1e-6
    neigh1 = jnp.dot(adj, x, preferred_element_type=jnp.float32) / deg
    h1 = jnp.maximum(
        jnp.dot(jnp.concatenate([x, neigh1], axis=1), w1_ref[...],
                preferred_element_type=jnp.float32), 0.0)
    neigh2 = jnp.dot(adj, h1, preferred_element_type=jnp.float32) / deg
    h2 = jnp.maximum(
        jnp.dot(jnp.concatenate([h1, neigh2], axis=1), w2_ref[...],
                preferred_element_type=jnp.float32), 0.0)
    out_ref[0] = h2


def _sage2(x, adj, w1, w2):
    b, n, d = x.shape
    h = w2.shape[1]
    return pl.pallas_call(
        _sage2_body,
        grid=(b,),
        in_specs=[
            pl.BlockSpec((1, n, d), lambda i: (i, 0, 0)),
            pl.BlockSpec((1, n, n), lambda i: (i, 0, 0)),
            pl.BlockSpec(w1.shape, lambda i: (0, 0)),
            pl.BlockSpec(w2.shape, lambda i: (0, 0)),
        ],
        out_specs=pl.BlockSpec((1, n, h), lambda i: (i, 0, 0)),
        out_shape=jax.ShapeDtypeStruct((b, n, h), jnp.float32),
    )(x, adj, w1, w2)


def _gather(h2_flat, idx):
    bn, d = h2_flat.shape
    mesh = plsc.VectorSubcoreMesh(core_axis_name="c", subcore_axis_name="s")

    @pl.kernel(out_type=jax.ShapeDtypeStruct((bn, d), h2_flat.dtype),
               mesh=mesh)
    def k(x_hbm, i_hbm, o_hbm):
        def body(i_vmem, o_vmem):
            pltpu.sync_copy(x_hbm.at[i_vmem.at[0]], o_vmem)

        pltpu.emit_pipeline(
            body,
            grid=(bn // _GATHER_WINDOW,),
            in_specs=[pl.BlockSpec((1, _GATHER_WINDOW), lambda i: (0, i))],
            out_specs=[pl.BlockSpec((_GATHER_WINDOW, d), lambda i: (i, 0))],
            core_axis_name=("c", "s"),
            dimension_semantics=(pltpu.PARALLEL,),
        )(i_hbm, o_hbm)

    return k(h2_flat, idx)


def _loss_body(emb_ref, adjm_ref, acc_ref, msum_ref):
    b = pl.program_id(0)
    r = pl.program_id(1)
    ef = emb_ref[0]
    er = emb_ref[0, pl.ds(r * _ROWS, _ROWS), :]
    z = jax.lax.dot_general(er, ef, (((1,), (1,)), ((), ())),
                            preferred_element_type=jnp.float32)
    a = adjm_ref[0]
    s = 1.0 / (1.0 + jnp.exp(-z))
    t = a * jnp.log(s + 1e-7) + (1.0 - a) * jnp.log(1.0000001192092896 - s)
    part = jnp.sum(t)

    @pl.when((b == 0) & (r == 0))
    def _():
        acc_ref[...] = jnp.zeros_like(acc_ref)
        msum_ref[...] = jnp.zeros_like(msum_ref)

    acc_ref[...] += part
    msum_ref[pl.ds(b, 1), :] += jnp.sum(er, axis=0, keepdims=True)


def _loss_call(embed, adjm):
    b, n, d = embed.shape
    return pl.pallas_call(
        _loss_body,
        grid=(b, n // _ROWS),
        in_specs=[
            pl.BlockSpec((1, n, d), lambda i, r: (i, 0, 0)),
            pl.BlockSpec((1, _ROWS, n), lambda i, r: (i, r, 0)),
        ],
        out_specs=(pl.BlockSpec((8, 128), lambda i, r: (0, 0)),
                   pl.BlockSpec((8, d), lambda i, r: (0, 0))),
        out_shape=(jax.ShapeDtypeStruct((8, 128), jnp.float32),
                   jax.ShapeDtypeStruct((8, d), jnp.float32)),
    )(embed, adjm)


def kernel(features, adj_lists, nodes, adj_matrixs, W1, W2,
           fc1_w, fc1_b, fc2_w, fc2_b, fc3_w, fc3_b):
    b, n, d = features.shape
    h = W1.shape[1]
    out_d = W2.shape[1]

    h2 = _sage2(features, adj_lists, W1, W2)

    idx = (nodes.astype(jnp.int32)
           + (jnp.arange(b, dtype=jnp.int32) * n)[:, None]).reshape(1, b * n)
    embed = _gather(h2.reshape(b * n, out_d), idx).reshape(b, n, out_d)

    acc, msums = _loss_call(embed, adj_matrixs)
    all_embeds = msums[:b] * (1.0 / n)
    all_recons_loss = acc[0, 0] * (-1.0 / (b * n * n))
    x = jax.nn.relu(all_embeds @ fc1_w + fc1_b)
    x = jax.nn.relu(x @ fc2_w + fc2_b)
    v = x @ fc3_w + fc3_b
    return (v, all_recons_loss)
